# baseline (speedup 1.0000x reference)
import functools

import jax
import jax.numpy as jnp
from jax import lax
from jax.experimental import pallas as pl
from jax.experimental.pallas import tpu as pltpu
from jax.experimental.pallas import tpu_sc as plsc

LANES = 128
NC, NS = 2, 16
NW = NC * NS


def _mesh():
    return plsc.VectorSubcoreMesh(core_axis_name="c", subcore_axis_name="s")


def _degree_kernel(n_pad, n_batches):
    @functools.partial(
        pl.kernel,
        mesh=_mesh(),
        out_type=jax.ShapeDtypeStruct((NC, n_pad), jnp.float32),
        scratch_types=[
            pltpu.VMEM((n_batches, LANES), jnp.int32),
            pltpu.VMEM((LANES,), jnp.float32),
            pltpu.VMEM_SHARED((n_pad,), jnp.float32),
        ],
    )
    def k(dst_hbm, zero1_hbm, out_hbm, dst_v, ones_v, acc):
        cid = lax.axis_index("c")
        sid = lax.axis_index("s")
        wid = sid * NC + cid

        @pl.when(sid == 0)
        def _():
            pltpu.sync_copy(zero1_hbm, acc)

        for i in range(LANES // 16):
            ones_v[pl.ds(16 * i, 16)] = jnp.ones((16,), jnp.float32)
        pltpu.sync_copy(dst_hbm.at[wid], dst_v)
        plsc.subcore_barrier()

        def body(b, carry):
            pltpu.sync_copy(ones_v, acc.at[dst_v.at[b]], add=True)
            return carry

        lax.fori_loop(0, n_batches, body, 0)
        plsc.subcore_barrier()

        @pl.when(sid == 0)
        def _():
            pltpu.sync_copy(acc, out_hbm.at[cid])

    return k


CHUNK = 16
NBUF = 4


def _spmm_kernel(n_pad, f, nc0, nc1, batch):
    rows_per = n_pad // NS

    @functools.partial(
        pl.kernel,
        mesh=_mesh(),
        out_type=jax.ShapeDtypeStruct((NC, n_pad, f), jnp.float32),
        scratch_types=[
            pltpu.VMEM((CHUNK, batch), jnp.int32),
            pltpu.VMEM((CHUNK, batch), jnp.int32),
        ] + [pltpu.VMEM((batch, f), jnp.float32) for _ in range(NBUF)]
          + [
            pltpu.VMEM_SHARED((n_pad, f), jnp.float32),
        ] + [pltpu.SemaphoreType.DMA for _ in range(NBUF)],
    )
    def k(src_hbm, dst_hbm, u_hbm, zero_hbm, out_hbm, src_v, dst_v, *rest):
        bufs = rest[:NBUF]
        acc = rest[NBUF]
        sems = rest[NBUF + 1:]
        cid = lax.axis_index("c")
        sid = lax.axis_index("s")
        wid = sid * NC + cid
        sl = pl.ds(sid * rows_per, rows_per)
        n_local = jnp.where(cid == 0, nc0, nc1)

        pltpu.sync_copy(zero_hbm.at[sl], acc.at[sl])
        plsc.subcore_barrier()

        def chunk_body(ci, carry):
            pltpu.sync_copy(src_hbm.at[wid, ci], src_v)
            pltpu.sync_copy(dst_hbm.at[wid, ci], dst_v)
            for j in range(NBUF):
                pltpu.async_copy(u_hbm.at[src_v.at[j]], bufs[j], sems[j])
            for g in range(0, CHUNK, NBUF):
                for j in range(NBUF):
                    b = g + j
                    pltpu.make_async_copy(
                        u_hbm.at[src_v.at[b]], bufs[j], sems[j]).wait()
                    pltpu.sync_copy(bufs[j], acc.at[dst_v.at[b]], add=True)
                    if b + NBUF < CHUNK:
                        pltpu.async_copy(
                            u_hbm.at[src_v.at[b + NBUF]], bufs[j], sems[j])
            return carry

        lax.fori_loop(0, n_local, chunk_body, 0)
        plsc.subcore_barrier()
        pltpu.sync_copy(acc.at[sl], out_hbm.at[cid, sl])

    return k


def _k2a_body(x_ref, w_ref, xw_ref):
    xw_ref[...] = jnp.dot(
        x_ref[...], w_ref[...], preferred_element_type=jnp.float32)


def _k2b_body(xw_ref, deg_ref, u1_ref, dis_ref):
    d = deg_ref[0] + deg_ref[1] + 1.0
    dis = lax.rsqrt(d)
    u1_ref[...] = xw_ref[...] * dis
    dis_ref[...] = dis


def _k4_body(t_ref, xw_ref, dis_ref, b_ref, h1_ref, u2_ref):
    t = t_ref[0] + t_ref[1]
    dis = dis_ref[...]
    agg = dis * t + (dis * dis) * xw_ref[...]
    h1 = jnp.maximum(agg + b_ref[...], 0.0)
    h1_ref[...] = h1
    u2_ref[...] = dis * h1


def _k6_body(t_ref, h_ref, dis_ref, w_ref, b_ref, o_ref):
    t = t_ref[0] + t_ref[1]
    dis = dis_ref[...]
    agg = dis * t + (dis * dis) * h_ref[...]
    o_ref[...] = (
        jnp.dot(agg, w_ref[...], preferred_element_type=jnp.float32) + b_ref[...]
    )


def kernel(x, edge_index, W1, b1, W2, b2):
    n, f = x.shape
    e = edge_index.shape[1]
    o = W2.shape[1]

    n_pad = 10240
    batch_sp = 64
    e_chunk = CHUNK * batch_sp
    n_chunks = -(-e // (NW * e_chunk))
    e_pad = NW * e_chunk * n_chunks
    per_tile_batches = e_pad // (NW * LANES)

    src = edge_index[0].astype(jnp.int32)
    dst = edge_index[1].astype(jnp.int32)
    fill = n + jnp.arange(e_pad - e, dtype=jnp.int32) % (n_pad - n)
    src_f = jnp.concatenate([src, fill])
    dst_f = jnp.concatenate([dst, fill])
    src_r = src_f.reshape(NW, per_tile_batches, LANES)
    dst_r = dst_f.reshape(NW, per_tile_batches, LANES)

    nc0, nc1 = 15, 5
    assert NS * (nc0 + nc1) * e_chunk >= e
    ncmax = max(nc0, nc1)
    q0, q1 = nc0 * e_chunk, nc1 * e_chunk
    f0 = NS * q0

    def _split(flat):
        parts = []
        pad1 = jnp.zeros((ncmax * e_chunk - q1,), jnp.int32)
        for w in range(NW):
            k = w // NC
            if w % NC == 0:
                parts.append(lax.dynamic_slice(flat, (k * q0,), (q0,)))
            else:
                parts.append(jnp.concatenate(
                    [lax.dynamic_slice(flat, (f0 + k * q1,), (q1,)), pad1]))
        return jnp.stack(parts).reshape(NW, ncmax, CHUNK, batch_sp)

    src_c = _split(src_f)
    dst_c = _split(dst_f)

    x_pad = jnp.pad(x, ((0, n_pad - n), (0, 0)))
    zero1 = jnp.zeros((n_pad,), jnp.float32)
    zero2 = jnp.zeros((n_pad, f), jnp.float32)

    deg_p = _degree_kernel(n_pad, per_tile_batches)(dst_r, zero1)
    deg3 = deg_p.reshape(NC, n_pad, 1)

    nb = 2
    blk = n_pad // nb
    xw = pl.pallas_call(
        _k2a_body,
        grid=(nb,),
        in_specs=[
            pl.BlockSpec((blk, f), lambda i: (i, 0)),
            pl.BlockSpec((f, f), lambda i: (0, 0)),
        ],
        out_specs=pl.BlockSpec((blk, f), lambda i: (i, 0)),
        out_shape=jax.ShapeDtypeStruct((n_pad, f), jnp.float32),
    )(x_pad, W1)

    u1, dis = pl.pallas_call(
        _k2b_body,
        grid=(nb,),
        in_specs=[
            pl.BlockSpec((blk, f), lambda i: (i, 0)),
            pl.BlockSpec((NC, blk, 1), lambda i: (0, i, 0)),
        ],
        out_specs=[
            pl.BlockSpec((blk, f), lambda i: (i, 0)),
            pl.BlockSpec((blk, 1), lambda i: (i, 0)),
        ],
        out_shape=[
            jax.ShapeDtypeStruct((n_pad, f), jnp.float32),
            jax.ShapeDtypeStruct((n_pad, 1), jnp.float32),
        ],
    )(xw, deg3)

    tmp1 = _spmm_kernel(n_pad, f, nc0, nc1, batch_sp)(src_c, dst_c, u1, zero2)

    h1, u2 = pl.pallas_call(
        _k4_body,
        grid=(nb,),
        in_specs=[
            pl.BlockSpec((NC, blk, f), lambda i: (0, i, 0)),
            pl.BlockSpec((blk, f), lambda i: (i, 0)),
            pl.BlockSpec((blk, 1), lambda i: (i, 0)),
            pl.BlockSpec((1, f), lambda i: (0, 0)),
        ],
        out_specs=[
            pl.BlockSpec((blk, f), lambda i: (i, 0)),
            pl.BlockSpec((blk, f), lambda i: (i, 0)),
        ],
        out_shape=[
            jax.ShapeDtypeStruct((n_pad, f), jnp.float32),
            jax.ShapeDtypeStruct((n_pad, f), jnp.float32),
        ],
    )(tmp1, xw, dis, b1.reshape(1, f))

    tmp2 = _spmm_kernel(n_pad, f, nc0, nc1, batch_sp)(src_c, dst_c, u2, zero2)

    ob = 2000
    og = n // ob
    out = pl.pallas_call(
        _k6_body,
        grid=(og,),
        in_specs=[
            pl.BlockSpec((NC, ob, f), lambda i: (0, i, 0)),
            pl.BlockSpec((ob, f), lambda i: (i, 0)),
            pl.BlockSpec((ob, 1), lambda i: (i, 0)),
            pl.BlockSpec((f, o), lambda i: (0, 0)),
            pl.BlockSpec((1, o), lambda i: (0, 0)),
        ],
        out_specs=pl.BlockSpec((ob, o), lambda i: (i, 0)),
        out_shape=jax.ShapeDtypeStruct((n, o), jnp.float32),
    )(tmp2, h1, dis, W2, b2.reshape(1, o))

    return out


# speedup vs baseline: 1.0269x; 1.0269x over previous
import functools

import jax
import jax.numpy as jnp
from jax import lax
from jax.experimental import pallas as pl
from jax.experimental.pallas import tpu as pltpu
from jax.experimental.pallas import tpu_sc as plsc

LANES = 128
NC, NS = 2, 16
NW = NC * NS


def _mesh():
    return plsc.VectorSubcoreMesh(core_axis_name="c", subcore_axis_name="s")


def _degree_kernel(n_pad, n_batches):
    @functools.partial(
        pl.kernel,
        mesh=_mesh(),
        out_type=jax.ShapeDtypeStruct((NC, n_pad), jnp.float32),
        scratch_types=[
            pltpu.VMEM((n_batches, LANES), jnp.int32),
            pltpu.VMEM((LANES,), jnp.float32),
            pltpu.VMEM_SHARED((n_pad,), jnp.float32),
        ],
    )
    def k(dst_hbm, zero1_hbm, out_hbm, dst_v, ones_v, acc):
        cid = lax.axis_index("c")
        sid = lax.axis_index("s")
        wid = sid * NC + cid

        @pl.when(sid == 0)
        def _():
            pltpu.sync_copy(zero1_hbm, acc)

        for i in range(LANES // 16):
            ones_v[pl.ds(16 * i, 16)] = jnp.ones((16,), jnp.float32)
        pltpu.sync_copy(dst_hbm.at[wid], dst_v)
        plsc.subcore_barrier()

        def body(b, carry):
            pltpu.sync_copy(ones_v, acc.at[dst_v.at[b]], add=True)
            return carry

        lax.fori_loop(0, n_batches, body, 0)
        plsc.subcore_barrier()

        @pl.when(sid == 0)
        def _():
            pltpu.sync_copy(acc, out_hbm.at[cid])

    return k


CHUNK = 16
NBUF = 4


def _spmm_kernel(n_pad, f, nc0, nc1, batch):
    rows_per = n_pad // NS

    @functools.partial(
        pl.kernel,
        mesh=_mesh(),
        out_type=jax.ShapeDtypeStruct((NC, n_pad, f), jnp.float32),
        scratch_types=[
            pltpu.VMEM((CHUNK, batch), jnp.int32),
            pltpu.VMEM((CHUNK, batch), jnp.int32),
        ] + [pltpu.VMEM((batch, f), jnp.float32) for _ in range(NBUF)]
          + [
            pltpu.VMEM_SHARED((n_pad, f), jnp.float32),
        ] + [pltpu.SemaphoreType.DMA for _ in range(NBUF)],
    )
    def k(src_hbm, dst_hbm, u_hbm, zero_hbm, out_hbm, src_v, dst_v, *rest):
        bufs = rest[:NBUF]
        acc = rest[NBUF]
        sems = rest[NBUF + 1:]
        cid = lax.axis_index("c")
        sid = lax.axis_index("s")
        wid = sid * NC + cid
        sl = pl.ds(sid * rows_per, rows_per)
        n_local = jnp.where(cid == 0, nc0, nc1)

        pltpu.sync_copy(zero_hbm.at[sl], acc.at[sl])
        plsc.subcore_barrier()

        def chunk_body(ci, carry):
            pltpu.sync_copy(src_hbm.at[wid, ci], src_v)
            pltpu.sync_copy(dst_hbm.at[wid, ci], dst_v)
            for j in range(NBUF):
                pltpu.async_copy(u_hbm.at[src_v.at[j]], bufs[j], sems[j])
            for g in range(0, CHUNK, NBUF):
                for j in range(NBUF):
                    b = g + j
                    pltpu.make_async_copy(
                        u_hbm.at[src_v.at[b]], bufs[j], sems[j]).wait()
                    pltpu.sync_copy(bufs[j], acc.at[dst_v.at[b]], add=True)
                    if b + NBUF < CHUNK:
                        pltpu.async_copy(
                            u_hbm.at[src_v.at[b + NBUF]], bufs[j], sems[j])
            return carry

        lax.fori_loop(0, n_local, chunk_body, 0)
        plsc.subcore_barrier()
        pltpu.sync_copy(acc.at[sl], out_hbm.at[cid, sl])

    return k


def _k2_body(x_ref, w_ref, deg_ref, xw_ref, u1_ref, dis_ref):
    xw = jnp.dot(x_ref[...], w_ref[...], preferred_element_type=jnp.float32)
    d = deg_ref[0] + deg_ref[1] + 1.0
    dis = lax.rsqrt(d)
    xw_ref[...] = xw
    u1_ref[...] = xw * dis
    dis_ref[...] = dis


def _k4_body(t_ref, xw_ref, dis_ref, b_ref, h1_ref, u2_ref):
    t = t_ref[0] + t_ref[1]
    dis = dis_ref[...]
    agg = dis * t + (dis * dis) * xw_ref[...]
    h1 = jnp.maximum(agg + b_ref[...], 0.0)
    h1_ref[...] = h1
    u2_ref[...] = dis * h1


def _k6_body(t_ref, h_ref, dis_ref, w_ref, b_ref, o_ref):
    t = t_ref[0] + t_ref[1]
    dis = dis_ref[...]
    agg = dis * t + (dis * dis) * h_ref[...]
    o_ref[...] = (
        jnp.dot(agg, w_ref[...], preferred_element_type=jnp.float32) + b_ref[...]
    )


def kernel(x, edge_index, W1, b1, W2, b2):
    n, f = x.shape
    e = edge_index.shape[1]
    o = W2.shape[1]

    n_pad = 10240
    batch_sp = 64
    e_chunk = CHUNK * batch_sp
    n_chunks = -(-e // (NW * e_chunk))
    e_pad = NW * e_chunk * n_chunks
    per_tile_batches = e_pad // (NW * LANES)

    src = edge_index[0].astype(jnp.int32)
    dst = edge_index[1].astype(jnp.int32)
    fill = n + jnp.arange(e_pad - e, dtype=jnp.int32) % (n_pad - n)
    src_f = jnp.concatenate([src, fill])
    dst_f = jnp.concatenate([dst, fill])
    src_r = src_f.reshape(NW, per_tile_batches, LANES)
    dst_r = dst_f.reshape(NW, per_tile_batches, LANES)

    nc0, nc1 = 15, 5
    assert NS * (nc0 + nc1) * e_chunk >= e
    ncmax = max(nc0, nc1)
    q0, q1 = nc0 * e_chunk, nc1 * e_chunk
    f0 = NS * q0

    def _split(flat):
        parts = []
        pad1 = jnp.zeros((ncmax * e_chunk - q1,), jnp.int32)
        for w in range(NW):
            k = w // NC
            if w % NC == 0:
                parts.append(lax.dynamic_slice(flat, (k * q0,), (q0,)))
            else:
                parts.append(jnp.concatenate(
                    [lax.dynamic_slice(flat, (f0 + k * q1,), (q1,)), pad1]))
        return jnp.stack(parts).reshape(NW, ncmax, CHUNK, batch_sp)

    src_c = _split(src_f)
    dst_c = _split(dst_f)

    x_pad = jnp.pad(x, ((0, n_pad - n), (0, 0)))
    zero1 = jnp.zeros((n_pad,), jnp.float32)
    zero2 = jnp.zeros((n_pad, f), jnp.float32)

    deg_p = _degree_kernel(n_pad, per_tile_batches)(dst_r, zero1)
    deg3 = deg_p.reshape(NC, n_pad, 1)

    nb = 4
    blk = n_pad // nb
    xw, u1, dis = pl.pallas_call(
        _k2_body,
        grid=(nb,),
        in_specs=[
            pl.BlockSpec((blk, f), lambda i: (i, 0)),
            pl.BlockSpec((f, f), lambda i: (0, 0)),
            pl.BlockSpec((NC, blk, 1), lambda i: (0, i, 0)),
        ],
        out_specs=[
            pl.BlockSpec((blk, f), lambda i: (i, 0)),
            pl.BlockSpec((blk, f), lambda i: (i, 0)),
            pl.BlockSpec((blk, 1), lambda i: (i, 0)),
        ],
        out_shape=[
            jax.ShapeDtypeStruct((n_pad, f), jnp.float32),
            jax.ShapeDtypeStruct((n_pad, f), jnp.float32),
            jax.ShapeDtypeStruct((n_pad, 1), jnp.float32),
        ],
    )(x_pad, W1, deg3)

    tmp1 = _spmm_kernel(n_pad, f, nc0, nc1, batch_sp)(src_c, dst_c, u1, zero2)

    h1, u2 = pl.pallas_call(
        _k4_body,
        grid=(nb,),
        in_specs=[
            pl.BlockSpec((NC, blk, f), lambda i: (0, i, 0)),
            pl.BlockSpec((blk, f), lambda i: (i, 0)),
            pl.BlockSpec((blk, 1), lambda i: (i, 0)),
            pl.BlockSpec((1, f), lambda i: (0, 0)),
        ],
        out_specs=[
            pl.BlockSpec((blk, f), lambda i: (i, 0)),
            pl.BlockSpec((blk, f), lambda i: (i, 0)),
        ],
        out_shape=[
            jax.ShapeDtypeStruct((n_pad, f), jnp.float32),
            jax.ShapeDtypeStruct((n_pad, f), jnp.float32),
        ],
    )(tmp1, xw, dis, b1.reshape(1, f))

    tmp2 = _spmm_kernel(n_pad, f, nc0, nc1, batch_sp)(src_c, dst_c, u2, zero2)

    ob = 2000
    og = n // ob
    out = pl.pallas_call(
        _k6_body,
        grid=(og,),
        in_specs=[
            pl.BlockSpec((NC, ob, f), lambda i: (0, i, 0)),
            pl.BlockSpec((ob, f), lambda i: (i, 0)),
            pl.BlockSpec((ob, 1), lambda i: (i, 0)),
            pl.BlockSpec((f, o), lambda i: (0, 0)),
            pl.BlockSpec((1, o), lambda i: (0, 0)),
        ],
        out_specs=pl.BlockSpec((ob, o), lambda i: (i, 0)),
        out_shape=jax.ShapeDtypeStruct((n, o), jnp.float32),
    )(tmp2, h1, dis, W2, b2.reshape(1, o))

    return out


# speedup vs baseline: 1.0717x; 1.0436x over previous
import functools

import jax
import jax.numpy as jnp
from jax import lax
from jax.experimental import pallas as pl
from jax.experimental.pallas import tpu as pltpu
from jax.experimental.pallas import tpu_sc as plsc

LANES = 128
NC, NS = 2, 16
NW = NC * NS


def _mesh():
    return plsc.VectorSubcoreMesh(core_axis_name="c", subcore_axis_name="s")


def _degree_kernel(n_pad, n_batches):
    @functools.partial(
        pl.kernel,
        mesh=_mesh(),
        out_type=jax.ShapeDtypeStruct((NC, n_pad), jnp.float32),
        scratch_types=[
            pltpu.VMEM((n_batches, LANES), jnp.int32),
            pltpu.VMEM((LANES,), jnp.float32),
            pltpu.VMEM_SHARED((n_pad,), jnp.float32),
        ],
    )
    def k(dst_hbm, zero1_hbm, out_hbm, dst_v, ones_v, acc):
        cid = lax.axis_index("c")
        sid = lax.axis_index("s")
        wid = sid * NC + cid

        @pl.when(sid == 0)
        def _():
            pltpu.sync_copy(zero1_hbm, acc)

        for i in range(LANES // 16):
            ones_v[pl.ds(16 * i, 16)] = jnp.ones((16,), jnp.float32)
        pltpu.sync_copy(dst_hbm.at[wid], dst_v)
        plsc.subcore_barrier()

        def body(b, carry):
            pltpu.sync_copy(ones_v, acc.at[dst_v.at[b]], add=True)
            return carry

        lax.fori_loop(0, n_batches, body, 0)
        plsc.subcore_barrier()

        @pl.when(sid == 0)
        def _():
            pltpu.sync_copy(acc, out_hbm.at[cid])

    return k


CHUNK = 16
NBUF = 4


def _spmm_kernel(n_pad, f, nc0, nc1, batch):
    rows_per = n_pad // NS

    @functools.partial(
        pl.kernel,
        mesh=_mesh(),
        out_type=jax.ShapeDtypeStruct((NC, n_pad, f), jnp.float32),
        scratch_types=[
            pltpu.VMEM((CHUNK, batch), jnp.int32),
            pltpu.VMEM((CHUNK, batch), jnp.int32),
        ] + [pltpu.VMEM((batch, f), jnp.float32) for _ in range(NBUF)]
          + [
            pltpu.VMEM_SHARED((n_pad, f), jnp.float32),
        ] + [pltpu.SemaphoreType.DMA for _ in range(NBUF)],
    )
    def k(src_hbm, dst_hbm, u_hbm, zero_hbm, out_hbm, src_v, dst_v, *rest):
        bufs = rest[:NBUF]
        acc = rest[NBUF]
        sems = rest[NBUF + 1:]
        cid = lax.axis_index("c")
        sid = lax.axis_index("s")
        wid = sid * NC + cid
        sl = pl.ds(sid * rows_per, rows_per)
        n_local = jnp.where(cid == 0, nc0, nc1)

        pltpu.sync_copy(zero_hbm.at[sl], acc.at[sl])
        plsc.subcore_barrier()

        def chunk_body(ci, carry):
            pltpu.sync_copy(src_hbm.at[wid, ci], src_v)
            pltpu.sync_copy(dst_hbm.at[wid, ci], dst_v)
            for j in range(NBUF):
                pltpu.async_copy(u_hbm.at[src_v.at[j]], bufs[j], sems[j])
            for g in range(0, CHUNK, NBUF):
                for j in range(NBUF):
                    b = g + j
                    pltpu.make_async_copy(
                        u_hbm.at[src_v.at[b]], bufs[j], sems[j]).wait()
                    pltpu.sync_copy(bufs[j], acc.at[dst_v.at[b]], add=True)
                    if b + NBUF < CHUNK:
                        pltpu.async_copy(
                            u_hbm.at[src_v.at[b + NBUF]], bufs[j], sems[j])
            return carry

        lax.fori_loop(0, n_local, chunk_body, 0)
        plsc.subcore_barrier()
        pltpu.sync_copy(acc.at[sl], out_hbm.at[cid, sl])

    return k


def _k2_body(x_ref, w_ref, deg_ref, xw_ref, u1_ref, dis_ref):
    xw = jnp.dot(x_ref[...], w_ref[...], preferred_element_type=jnp.float32)
    d = deg_ref[0] + deg_ref[1] + 1.0
    dis = lax.rsqrt(d)
    xw_ref[...] = xw
    u1_ref[...] = xw * dis
    dis_ref[...] = dis


def _k4_body(t_ref, xw_ref, dis_ref, b_ref, h1_ref, u2_ref):
    t = t_ref[0] + t_ref[1]
    dis = dis_ref[...]
    agg = dis * t + (dis * dis) * xw_ref[...]
    h1 = jnp.maximum(agg + b_ref[...], 0.0)
    h1_ref[...] = h1
    u2_ref[...] = dis * h1


def _k6_body(t_ref, h_ref, dis_ref, w_ref, b_ref, o_ref):
    t = t_ref[0] + t_ref[1]
    dis = dis_ref[...]
    agg = dis * t + (dis * dis) * h_ref[...]
    o_ref[...] = (
        jnp.dot(agg, w_ref[...], preferred_element_type=jnp.float32) + b_ref[...]
    )


def kernel(x, edge_index, W1, b1, W2, b2):
    n, f = x.shape
    e = edge_index.shape[1]
    o = W2.shape[1]

    n_pad = 10240
    batch_sp = 64
    e_chunk = CHUNK * batch_sp
    n_chunks = -(-e // (NW * e_chunk))
    e_pad = NW * e_chunk * n_chunks
    per_tile_batches = e_pad // (NW * LANES)

    src = edge_index[0].astype(jnp.int32)
    dst = edge_index[1].astype(jnp.int32)
    fill = n + jnp.arange(e_pad - e, dtype=jnp.int32) % (n_pad - n)
    src_f = jnp.concatenate([src, fill])
    dst_f = jnp.concatenate([dst, fill])
    src_r = src_f.reshape(NW, per_tile_batches, LANES)
    dst_r = dst_f.reshape(NW, per_tile_batches, LANES)

    nc0, nc1 = 14, 6
    assert NS * (nc0 + nc1) * e_chunk >= e
    ncmax = max(nc0, nc1)
    q0, q1 = nc0 * e_chunk, nc1 * e_chunk
    f0 = NS * q0

    def _split(flat):
        parts = []
        pad1 = jnp.zeros((ncmax * e_chunk - q1,), jnp.int32)
        for w in range(NW):
            k = w // NC
            if w % NC == 0:
                parts.append(lax.dynamic_slice(flat, (k * q0,), (q0,)))
            else:
                parts.append(jnp.concatenate(
                    [lax.dynamic_slice(flat, (f0 + k * q1,), (q1,)), pad1]))
        return jnp.stack(parts).reshape(NW, ncmax, CHUNK, batch_sp)

    src_c = _split(src_f)
    dst_c = _split(dst_f)

    x_pad = jnp.pad(x, ((0, n_pad - n), (0, 0)))
    zero1 = jnp.zeros((n_pad,), jnp.float32)
    zero2 = jnp.zeros((n_pad, f), jnp.float32)

    deg_p = _degree_kernel(n_pad, per_tile_batches)(dst_r, zero1)
    deg3 = deg_p.reshape(NC, n_pad, 1)

    nb = 4
    blk = n_pad // nb
    xw, u1, dis = pl.pallas_call(
        _k2_body,
        grid=(nb,),
        in_specs=[
            pl.BlockSpec((blk, f), lambda i: (i, 0)),
            pl.BlockSpec((f, f), lambda i: (0, 0)),
            pl.BlockSpec((NC, blk, 1), lambda i: (0, i, 0)),
        ],
        out_specs=[
            pl.BlockSpec((blk, f), lambda i: (i, 0)),
            pl.BlockSpec((blk, f), lambda i: (i, 0)),
            pl.BlockSpec((blk, 1), lambda i: (i, 0)),
        ],
        out_shape=[
            jax.ShapeDtypeStruct((n_pad, f), jnp.float32),
            jax.ShapeDtypeStruct((n_pad, f), jnp.float32),
            jax.ShapeDtypeStruct((n_pad, 1), jnp.float32),
        ],
    )(x_pad, W1, deg3)

    tmp1 = _spmm_kernel(n_pad, f, nc0, nc1, batch_sp)(src_c, dst_c, u1, zero2)

    h1, u2 = pl.pallas_call(
        _k4_body,
        grid=(nb,),
        in_specs=[
            pl.BlockSpec((NC, blk, f), lambda i: (0, i, 0)),
            pl.BlockSpec((blk, f), lambda i: (i, 0)),
            pl.BlockSpec((blk, 1), lambda i: (i, 0)),
            pl.BlockSpec((1, f), lambda i: (0, 0)),
        ],
        out_specs=[
            pl.BlockSpec((blk, f), lambda i: (i, 0)),
            pl.BlockSpec((blk, f), lambda i: (i, 0)),
        ],
        out_shape=[
            jax.ShapeDtypeStruct((n_pad, f), jnp.float32),
            jax.ShapeDtypeStruct((n_pad, f), jnp.float32),
        ],
    )(tmp1, xw, dis, b1.reshape(1, f))

    tmp2 = _spmm_kernel(n_pad, f, nc0, nc1, batch_sp)(src_c, dst_c, u2, zero2)

    ob = 2000
    og = n // ob
    out = pl.pallas_call(
        _k6_body,
        grid=(og,),
        in_specs=[
            pl.BlockSpec((NC, ob, f), lambda i: (0, i, 0)),
            pl.BlockSpec((ob, f), lambda i: (i, 0)),
            pl.BlockSpec((ob, 1), lambda i: (i, 0)),
            pl.BlockSpec((f, o), lambda i: (0, 0)),
            pl.BlockSpec((1, o), lambda i: (0, 0)),
        ],
        out_specs=pl.BlockSpec((ob, o), lambda i: (i, 0)),
        out_shape=jax.ShapeDtypeStruct((n, o), jnp.float32),
    )(tmp2, h1, dis, W2, b2.reshape(1, o))

    return out


# speedup vs baseline: 1.1263x; 1.0510x over previous
import functools

import jax
import jax.numpy as jnp
from jax import lax
from jax.experimental import pallas as pl
from jax.experimental.pallas import tpu as pltpu
from jax.experimental.pallas import tpu_sc as plsc

LANES = 128
NC, NS = 2, 16
NW = NC * NS


def _mesh():
    return plsc.VectorSubcoreMesh(core_axis_name="c", subcore_axis_name="s")


def _degree_kernel(n_pad, n_batches):
    @functools.partial(
        pl.kernel,
        mesh=_mesh(),
        out_type=jax.ShapeDtypeStruct((NC, n_pad), jnp.float32),
        scratch_types=[
            pltpu.VMEM((n_batches, LANES), jnp.int32),
            pltpu.VMEM((LANES,), jnp.float32),
            pltpu.VMEM_SHARED((n_pad,), jnp.float32),
        ],
    )
    def k(dst_hbm, zero1_hbm, out_hbm, dst_v, ones_v, acc):
        cid = lax.axis_index("c")
        sid = lax.axis_index("s")
        wid = sid * NC + cid

        @pl.when(sid == 0)
        def _():
            pltpu.sync_copy(zero1_hbm, acc)

        for i in range(LANES // 16):
            ones_v[pl.ds(16 * i, 16)] = jnp.ones((16,), jnp.float32)
        pltpu.sync_copy(dst_hbm.at[wid], dst_v)
        plsc.subcore_barrier()

        def body(b, carry):
            pltpu.sync_copy(ones_v, acc.at[dst_v.at[b]], add=True)
            return carry

        lax.fori_loop(0, n_batches, body, 0)
        plsc.subcore_barrier()

        @pl.when(sid == 0)
        def _():
            pltpu.sync_copy(acc, out_hbm.at[cid])

    return k


CHUNK = 16
NBUF = 4


def _spmm_kernel(n_pad, f, nc0, nc1, batch):
    rows_per = n_pad // NS

    @functools.partial(
        pl.kernel,
        mesh=_mesh(),
        out_type=jax.ShapeDtypeStruct((NC, n_pad, f), jnp.float32),
        scratch_types=[
            pltpu.VMEM((CHUNK, batch), jnp.int32),
            pltpu.VMEM((CHUNK, batch), jnp.int32),
        ] + [pltpu.VMEM((batch, f), jnp.float32) for _ in range(NBUF)]
          + [
            pltpu.VMEM_SHARED((n_pad, f), jnp.float32),
        ] + [pltpu.SemaphoreType.DMA for _ in range(NBUF)],
    )
    def k(src_hbm, dst_hbm, u_hbm, zero_hbm, out_hbm, src_v, dst_v, *rest):
        bufs = rest[:NBUF]
        acc = rest[NBUF]
        sems = rest[NBUF + 1:]
        cid = lax.axis_index("c")
        sid = lax.axis_index("s")
        wid = sid * NC + cid
        sl = pl.ds(sid * rows_per, rows_per)
        n_local = jnp.where(cid == 0, nc0, nc1)

        pltpu.sync_copy(zero_hbm.at[sl], acc.at[sl])
        plsc.subcore_barrier()

        def chunk_body(ci, carry):
            pltpu.sync_copy(src_hbm.at[wid, ci], src_v)
            pltpu.sync_copy(dst_hbm.at[wid, ci], dst_v)
            for j in range(NBUF):
                pltpu.async_copy(u_hbm.at[src_v.at[j]], bufs[j], sems[j])
            for g in range(0, CHUNK, NBUF):
                for j in range(NBUF):
                    b = g + j
                    pltpu.make_async_copy(
                        u_hbm.at[src_v.at[b]], bufs[j], sems[j]).wait()
                    pltpu.sync_copy(bufs[j], acc.at[dst_v.at[b]], add=True)
                    if b + NBUF < CHUNK:
                        pltpu.async_copy(
                            u_hbm.at[src_v.at[b + NBUF]], bufs[j], sems[j])
            return carry

        lax.fori_loop(0, n_local, chunk_body, 0)
        plsc.subcore_barrier()
        pltpu.sync_copy(acc.at[sl], out_hbm.at[cid, sl])

    return k


def _k2_body(x_ref, w_ref, deg_ref, xw_ref, u1_ref, dis_ref):
    xw = jnp.dot(x_ref[...], w_ref[...], preferred_element_type=jnp.float32)
    d = deg_ref[0] + deg_ref[1] + 1.0
    dis = lax.rsqrt(d)
    xw_ref[...] = xw
    u1_ref[...] = xw * dis
    dis_ref[...] = dis


def _k4_body(t_ref, xw_ref, dis_ref, b_ref, h1_ref, u2_ref):
    t = t_ref[0] + t_ref[1]
    dis = dis_ref[...]
    agg = dis * t + (dis * dis) * xw_ref[...]
    h1 = jnp.maximum(agg + b_ref[...], 0.0)
    h1_ref[...] = h1
    u2_ref[...] = dis * h1


def _k6_body(t_ref, h_ref, dis_ref, w_ref, b_ref, o_ref):
    t = t_ref[0] + t_ref[1]
    dis = dis_ref[...]
    agg = dis * t + (dis * dis) * h_ref[...]
    o_ref[...] = (
        jnp.dot(agg, w_ref[...], preferred_element_type=jnp.float32) + b_ref[...]
    )


def kernel(x, edge_index, W1, b1, W2, b2):
    n, f = x.shape
    e = edge_index.shape[1]
    o = W2.shape[1]

    n_pad = 10240
    batch_sp = 64
    e_chunk = CHUNK * batch_sp
    n_chunks = -(-e // (NW * e_chunk))
    e_pad = NW * e_chunk * n_chunks
    per_tile_batches = e_pad // (NW * LANES)

    src = edge_index[0].astype(jnp.int32)
    dst = edge_index[1].astype(jnp.int32)
    fill = n + jnp.arange(e_pad - e, dtype=jnp.int32) % (n_pad - n)
    src_f = jnp.concatenate([src, fill])
    dst_f = jnp.concatenate([dst, fill])
    src_r = src_f.reshape(NW, per_tile_batches, LANES)
    dst_r = dst_f.reshape(NW, per_tile_batches, LANES)

    nc0, nc1 = 13, 7
    assert NS * (nc0 + nc1) * e_chunk >= e
    ncmax = max(nc0, nc1)
    q0, q1 = nc0 * e_chunk, nc1 * e_chunk
    f0 = NS * q0

    def _split(flat):
        parts = []
        pad1 = jnp.zeros((ncmax * e_chunk - q1,), jnp.int32)
        for w in range(NW):
            k = w // NC
            if w % NC == 0:
                parts.append(lax.dynamic_slice(flat, (k * q0,), (q0,)))
            else:
                parts.append(jnp.concatenate(
                    [lax.dynamic_slice(flat, (f0 + k * q1,), (q1,)), pad1]))
        return jnp.stack(parts).reshape(NW, ncmax, CHUNK, batch_sp)

    src_c = _split(src_f)
    dst_c = _split(dst_f)

    x_pad = jnp.pad(x, ((0, n_pad - n), (0, 0)))
    zero1 = jnp.zeros((n_pad,), jnp.float32)
    zero2 = jnp.zeros((n_pad, f), jnp.float32)

    deg_p = _degree_kernel(n_pad, per_tile_batches)(dst_r, zero1)
    deg3 = deg_p.reshape(NC, n_pad, 1)

    nb = 4
    blk = n_pad // nb
    xw, u1, dis = pl.pallas_call(
        _k2_body,
        grid=(nb,),
        in_specs=[
            pl.BlockSpec((blk, f), lambda i: (i, 0)),
            pl.BlockSpec((f, f), lambda i: (0, 0)),
            pl.BlockSpec((NC, blk, 1), lambda i: (0, i, 0)),
        ],
        out_specs=[
            pl.BlockSpec((blk, f), lambda i: (i, 0)),
            pl.BlockSpec((blk, f), lambda i: (i, 0)),
            pl.BlockSpec((blk, 1), lambda i: (i, 0)),
        ],
        out_shape=[
            jax.ShapeDtypeStruct((n_pad, f), jnp.float32),
            jax.ShapeDtypeStruct((n_pad, f), jnp.float32),
            jax.ShapeDtypeStruct((n_pad, 1), jnp.float32),
        ],
    )(x_pad, W1, deg3)

    tmp1 = _spmm_kernel(n_pad, f, nc0, nc1, batch_sp)(src_c, dst_c, u1, zero2)

    h1, u2 = pl.pallas_call(
        _k4_body,
        grid=(nb,),
        in_specs=[
            pl.BlockSpec((NC, blk, f), lambda i: (0, i, 0)),
            pl.BlockSpec((blk, f), lambda i: (i, 0)),
            pl.BlockSpec((blk, 1), lambda i: (i, 0)),
            pl.BlockSpec((1, f), lambda i: (0, 0)),
        ],
        out_specs=[
            pl.BlockSpec((blk, f), lambda i: (i, 0)),
            pl.BlockSpec((blk, f), lambda i: (i, 0)),
        ],
        out_shape=[
            jax.ShapeDtypeStruct((n_pad, f), jnp.float32),
            jax.ShapeDtypeStruct((n_pad, f), jnp.float32),
        ],
    )(tmp1, xw, dis, b1.reshape(1, f))

    tmp2 = _spmm_kernel(n_pad, f, nc0, nc1, batch_sp)(src_c, dst_c, u2, zero2)

    ob = 2000
    og = n // ob
    out = pl.pallas_call(
        _k6_body,
        grid=(og,),
        in_specs=[
            pl.BlockSpec((NC, ob, f), lambda i: (0, i, 0)),
            pl.BlockSpec((ob, f), lambda i: (i, 0)),
            pl.BlockSpec((ob, 1), lambda i: (i, 0)),
            pl.BlockSpec((f, o), lambda i: (0, 0)),
            pl.BlockSpec((1, o), lambda i: (0, 0)),
        ],
        out_specs=pl.BlockSpec((ob, o), lambda i: (i, 0)),
        out_shape=jax.ShapeDtypeStruct((n, o), jnp.float32),
    )(tmp2, h1, dis, W2, b2.reshape(1, o))

    return out


# speedup vs baseline: 1.1830x; 1.0503x over previous
import functools

import jax
import jax.numpy as jnp
from jax import lax
from jax.experimental import pallas as pl
from jax.experimental.pallas import tpu as pltpu
from jax.experimental.pallas import tpu_sc as plsc

LANES = 128
NC, NS = 2, 16
NW = NC * NS


def _mesh():
    return plsc.VectorSubcoreMesh(core_axis_name="c", subcore_axis_name="s")


def _degree_kernel(n_pad, n_batches):
    @functools.partial(
        pl.kernel,
        mesh=_mesh(),
        out_type=jax.ShapeDtypeStruct((NC, n_pad), jnp.float32),
        scratch_types=[
            pltpu.VMEM((n_batches, LANES), jnp.int32),
            pltpu.VMEM((LANES,), jnp.float32),
            pltpu.VMEM_SHARED((n_pad,), jnp.float32),
        ],
    )
    def k(dst_hbm, zero1_hbm, out_hbm, dst_v, ones_v, acc):
        cid = lax.axis_index("c")
        sid = lax.axis_index("s")
        wid = sid * NC + cid

        @pl.when(sid == 0)
        def _():
            pltpu.sync_copy(zero1_hbm, acc)

        for i in range(LANES // 16):
            ones_v[pl.ds(16 * i, 16)] = jnp.ones((16,), jnp.float32)
        pltpu.sync_copy(dst_hbm.at[wid], dst_v)
        plsc.subcore_barrier()

        def body(b, carry):
            pltpu.sync_copy(ones_v, acc.at[dst_v.at[b]], add=True)
            return carry

        lax.fori_loop(0, n_batches, body, 0)
        plsc.subcore_barrier()

        @pl.when(sid == 0)
        def _():
            pltpu.sync_copy(acc, out_hbm.at[cid])

    return k


CHUNK = 16
NBUF = 4


def _spmm_kernel(n_pad, f, nc0, nc1, batch):
    rows_per = n_pad // NS

    @functools.partial(
        pl.kernel,
        mesh=_mesh(),
        out_type=jax.ShapeDtypeStruct((NC, n_pad, f), jnp.float32),
        scratch_types=[
            pltpu.VMEM((CHUNK, batch), jnp.int32),
            pltpu.VMEM((CHUNK, batch), jnp.int32),
        ] + [pltpu.VMEM((batch, f), jnp.float32) for _ in range(NBUF)]
          + [
            pltpu.VMEM_SHARED((n_pad, f), jnp.float32),
        ] + [pltpu.SemaphoreType.DMA for _ in range(NBUF)],
    )
    def k(src_hbm, dst_hbm, u_hbm, zero_hbm, out_hbm, src_v, dst_v, *rest):
        bufs = rest[:NBUF]
        acc = rest[NBUF]
        sems = rest[NBUF + 1:]
        cid = lax.axis_index("c")
        sid = lax.axis_index("s")
        wid = sid * NC + cid
        sl = pl.ds(sid * rows_per, rows_per)
        n_local = jnp.where(cid == 0, nc0, nc1)

        pltpu.sync_copy(zero_hbm.at[sl], acc.at[sl])
        plsc.subcore_barrier()

        def chunk_body(ci, carry):
            pltpu.sync_copy(src_hbm.at[wid, ci], src_v)
            pltpu.sync_copy(dst_hbm.at[wid, ci], dst_v)
            for j in range(NBUF):
                pltpu.async_copy(u_hbm.at[src_v.at[j]], bufs[j], sems[j])
            for g in range(0, CHUNK, NBUF):
                for j in range(NBUF):
                    b = g + j
                    pltpu.make_async_copy(
                        u_hbm.at[src_v.at[b]], bufs[j], sems[j]).wait()
                    pltpu.sync_copy(bufs[j], acc.at[dst_v.at[b]], add=True)
                    if b + NBUF < CHUNK:
                        pltpu.async_copy(
                            u_hbm.at[src_v.at[b + NBUF]], bufs[j], sems[j])
            return carry

        lax.fori_loop(0, n_local, chunk_body, 0)
        plsc.subcore_barrier()
        pltpu.sync_copy(acc.at[sl], out_hbm.at[cid, sl])

    return k


def _k2_body(x_ref, w_ref, deg_ref, xw_ref, u1_ref, dis_ref):
    xw = jnp.dot(x_ref[...], w_ref[...], preferred_element_type=jnp.float32)
    d = deg_ref[0] + deg_ref[1] + 1.0
    dis = lax.rsqrt(d)
    xw_ref[...] = xw
    u1_ref[...] = xw * dis
    dis_ref[...] = dis


def _k4_body(t_ref, xw_ref, dis_ref, b_ref, h1_ref, u2_ref):
    t = t_ref[0] + t_ref[1]
    dis = dis_ref[...]
    agg = dis * t + (dis * dis) * xw_ref[...]
    h1 = jnp.maximum(agg + b_ref[...], 0.0)
    h1_ref[...] = h1
    u2_ref[...] = dis * h1


def _k6_body(t_ref, h_ref, dis_ref, w_ref, b_ref, o_ref):
    t = t_ref[0] + t_ref[1]
    dis = dis_ref[...]
    agg = dis * t + (dis * dis) * h_ref[...]
    o_ref[...] = (
        jnp.dot(agg, w_ref[...], preferred_element_type=jnp.float32) + b_ref[...]
    )


def kernel(x, edge_index, W1, b1, W2, b2):
    n, f = x.shape
    e = edge_index.shape[1]
    o = W2.shape[1]

    n_pad = 10240
    batch_sp = 64
    e_chunk = CHUNK * batch_sp
    n_chunks = -(-e // (NW * e_chunk))
    e_pad = NW * e_chunk * n_chunks
    per_tile_batches = e_pad // (NW * LANES)

    src = edge_index[0].astype(jnp.int32)
    dst = edge_index[1].astype(jnp.int32)
    fill = n + jnp.arange(e_pad - e, dtype=jnp.int32) % (n_pad - n)
    src_f = jnp.concatenate([src, fill])
    dst_f = jnp.concatenate([dst, fill])
    src_r = src_f.reshape(NW, per_tile_batches, LANES)
    dst_r = dst_f.reshape(NW, per_tile_batches, LANES)

    nc0, nc1 = 12, 8
    assert NS * (nc0 + nc1) * e_chunk >= e
    ncmax = max(nc0, nc1)
    q0, q1 = nc0 * e_chunk, nc1 * e_chunk
    f0 = NS * q0

    def _split(flat):
        parts = []
        pad1 = jnp.zeros((ncmax * e_chunk - q1,), jnp.int32)
        for w in range(NW):
            k = w // NC
            if w % NC == 0:
                parts.append(lax.dynamic_slice(flat, (k * q0,), (q0,)))
            else:
                parts.append(jnp.concatenate(
                    [lax.dynamic_slice(flat, (f0 + k * q1,), (q1,)), pad1]))
        return jnp.stack(parts).reshape(NW, ncmax, CHUNK, batch_sp)

    src_c = _split(src_f)
    dst_c = _split(dst_f)

    x_pad = jnp.pad(x, ((0, n_pad - n), (0, 0)))
    zero1 = jnp.zeros((n_pad,), jnp.float32)
    zero2 = jnp.zeros((n_pad, f), jnp.float32)

    deg_p = _degree_kernel(n_pad, per_tile_batches)(dst_r, zero1)
    deg3 = deg_p.reshape(NC, n_pad, 1)

    nb = 4
    blk = n_pad // nb
    xw, u1, dis = pl.pallas_call(
        _k2_body,
        grid=(nb,),
        in_specs=[
            pl.BlockSpec((blk, f), lambda i: (i, 0)),
            pl.BlockSpec((f, f), lambda i: (0, 0)),
            pl.BlockSpec((NC, blk, 1), lambda i: (0, i, 0)),
        ],
        out_specs=[
            pl.BlockSpec((blk, f), lambda i: (i, 0)),
            pl.BlockSpec((blk, f), lambda i: (i, 0)),
            pl.BlockSpec((blk, 1), lambda i: (i, 0)),
        ],
        out_shape=[
            jax.ShapeDtypeStruct((n_pad, f), jnp.float32),
            jax.ShapeDtypeStruct((n_pad, f), jnp.float32),
            jax.ShapeDtypeStruct((n_pad, 1), jnp.float32),
        ],
    )(x_pad, W1, deg3)

    tmp1 = _spmm_kernel(n_pad, f, nc0, nc1, batch_sp)(src_c, dst_c, u1, zero2)

    h1, u2 = pl.pallas_call(
        _k4_body,
        grid=(nb,),
        in_specs=[
            pl.BlockSpec((NC, blk, f), lambda i: (0, i, 0)),
            pl.BlockSpec((blk, f), lambda i: (i, 0)),
            pl.BlockSpec((blk, 1), lambda i: (i, 0)),
            pl.BlockSpec((1, f), lambda i: (0, 0)),
        ],
        out_specs=[
            pl.BlockSpec((blk, f), lambda i: (i, 0)),
            pl.BlockSpec((blk, f), lambda i: (i, 0)),
        ],
        out_shape=[
            jax.ShapeDtypeStruct((n_pad, f), jnp.float32),
            jax.ShapeDtypeStruct((n_pad, f), jnp.float32),
        ],
    )(tmp1, xw, dis, b1.reshape(1, f))

    tmp2 = _spmm_kernel(n_pad, f, nc0, nc1, batch_sp)(src_c, dst_c, u2, zero2)

    ob = 2000
    og = n // ob
    out = pl.pallas_call(
        _k6_body,
        grid=(og,),
        in_specs=[
            pl.BlockSpec((NC, ob, f), lambda i: (0, i, 0)),
            pl.BlockSpec((ob, f), lambda i: (i, 0)),
            pl.BlockSpec((ob, 1), lambda i: (i, 0)),
            pl.BlockSpec((f, o), lambda i: (0, 0)),
            pl.BlockSpec((1, o), lambda i: (0, 0)),
        ],
        out_specs=pl.BlockSpec((ob, o), lambda i: (i, 0)),
        out_shape=jax.ShapeDtypeStruct((n, o), jnp.float32),
    )(tmp2, h1, dis, W2, b2.reshape(1, o))

    return out


# speedup vs baseline: 1.2478x; 1.0548x over previous
import functools

import jax
import jax.numpy as jnp
from jax import lax
from jax.experimental import pallas as pl
from jax.experimental.pallas import tpu as pltpu
from jax.experimental.pallas import tpu_sc as plsc

LANES = 128
NC, NS = 2, 16
NW = NC * NS


def _mesh():
    return plsc.VectorSubcoreMesh(core_axis_name="c", subcore_axis_name="s")


def _degree_kernel(n_pad, n_batches):
    @functools.partial(
        pl.kernel,
        mesh=_mesh(),
        out_type=jax.ShapeDtypeStruct((NC, n_pad), jnp.float32),
        scratch_types=[
            pltpu.VMEM((n_batches, LANES), jnp.int32),
            pltpu.VMEM((LANES,), jnp.float32),
            pltpu.VMEM_SHARED((n_pad,), jnp.float32),
        ],
    )
    def k(dst_hbm, zero1_hbm, out_hbm, dst_v, ones_v, acc):
        cid = lax.axis_index("c")
        sid = lax.axis_index("s")
        wid = sid * NC + cid

        @pl.when(sid == 0)
        def _():
            pltpu.sync_copy(zero1_hbm, acc)

        for i in range(LANES // 16):
            ones_v[pl.ds(16 * i, 16)] = jnp.ones((16,), jnp.float32)
        pltpu.sync_copy(dst_hbm.at[wid], dst_v)
        plsc.subcore_barrier()

        def body(b, carry):
            pltpu.sync_copy(ones_v, acc.at[dst_v.at[b]], add=True)
            return carry

        lax.fori_loop(0, n_batches, body, 0)
        plsc.subcore_barrier()

        @pl.when(sid == 0)
        def _():
            pltpu.sync_copy(acc, out_hbm.at[cid])

    return k


CHUNK = 16
NBUF = 4


def _spmm_kernel(n_pad, f, nc0, nc1, batch):
    rows_per = n_pad // NS

    @functools.partial(
        pl.kernel,
        mesh=_mesh(),
        out_type=jax.ShapeDtypeStruct((NC, n_pad, f), jnp.float32),
        scratch_types=[
            pltpu.VMEM((CHUNK, batch), jnp.int32),
            pltpu.VMEM((CHUNK, batch), jnp.int32),
        ] + [pltpu.VMEM((batch, f), jnp.float32) for _ in range(NBUF)]
          + [
            pltpu.VMEM_SHARED((n_pad, f), jnp.float32),
        ] + [pltpu.SemaphoreType.DMA for _ in range(NBUF)],
    )
    def k(src_hbm, dst_hbm, u_hbm, zero_hbm, out_hbm, src_v, dst_v, *rest):
        bufs = rest[:NBUF]
        acc = rest[NBUF]
        sems = rest[NBUF + 1:]
        cid = lax.axis_index("c")
        sid = lax.axis_index("s")
        wid = sid * NC + cid
        sl = pl.ds(sid * rows_per, rows_per)
        n_local = jnp.where(cid == 0, nc0, nc1)

        pltpu.sync_copy(zero_hbm.at[sl], acc.at[sl])
        plsc.subcore_barrier()

        def chunk_body(ci, carry):
            pltpu.sync_copy(src_hbm.at[wid, ci], src_v)
            pltpu.sync_copy(dst_hbm.at[wid, ci], dst_v)
            for j in range(NBUF):
                pltpu.async_copy(u_hbm.at[src_v.at[j]], bufs[j], sems[j])
            for g in range(0, CHUNK, NBUF):
                for j in range(NBUF):
                    b = g + j
                    pltpu.make_async_copy(
                        u_hbm.at[src_v.at[b]], bufs[j], sems[j]).wait()
                    pltpu.sync_copy(bufs[j], acc.at[dst_v.at[b]], add=True)
                    if b + NBUF < CHUNK:
                        pltpu.async_copy(
                            u_hbm.at[src_v.at[b + NBUF]], bufs[j], sems[j])
            return carry

        lax.fori_loop(0, n_local, chunk_body, 0)
        plsc.subcore_barrier()
        pltpu.sync_copy(acc.at[sl], out_hbm.at[cid, sl])

    return k


def _k2_body(x_ref, w_ref, deg_ref, xw_ref, u1_ref, dis_ref):
    xw = jnp.dot(x_ref[...], w_ref[...], preferred_element_type=jnp.float32)
    d = deg_ref[0] + deg_ref[1] + 1.0
    dis = lax.rsqrt(d)
    xw_ref[...] = xw
    u1_ref[...] = xw * dis
    dis_ref[...] = dis


def _k4_body(t_ref, xw_ref, dis_ref, b_ref, h1_ref, u2_ref):
    t = t_ref[0] + t_ref[1]
    dis = dis_ref[...]
    agg = dis * t + (dis * dis) * xw_ref[...]
    h1 = jnp.maximum(agg + b_ref[...], 0.0)
    h1_ref[...] = h1
    u2_ref[...] = dis * h1


def _k6_body(t_ref, h_ref, dis_ref, w_ref, b_ref, o_ref):
    t = t_ref[0] + t_ref[1]
    dis = dis_ref[...]
    agg = dis * t + (dis * dis) * h_ref[...]
    o_ref[...] = (
        jnp.dot(agg, w_ref[...], preferred_element_type=jnp.float32) + b_ref[...]
    )


def kernel(x, edge_index, W1, b1, W2, b2):
    n, f = x.shape
    e = edge_index.shape[1]
    o = W2.shape[1]

    n_pad = 10240
    batch_sp = 64
    e_chunk = CHUNK * batch_sp
    n_chunks = -(-e // (NW * e_chunk))
    e_pad = NW * e_chunk * n_chunks
    per_tile_batches = e_pad // (NW * LANES)

    src = edge_index[0].astype(jnp.int32)
    dst = edge_index[1].astype(jnp.int32)
    fill = n + jnp.arange(e_pad - e, dtype=jnp.int32) % (n_pad - n)
    src_f = jnp.concatenate([src, fill])
    dst_f = jnp.concatenate([dst, fill])
    src_r = src_f.reshape(NW, per_tile_batches, LANES)
    dst_r = dst_f.reshape(NW, per_tile_batches, LANES)

    nc0, nc1 = 11, 9
    assert NS * (nc0 + nc1) * e_chunk >= e
    ncmax = max(nc0, nc1)
    q0, q1 = nc0 * e_chunk, nc1 * e_chunk
    f0 = NS * q0

    def _split(flat):
        parts = []
        pad1 = jnp.zeros((ncmax * e_chunk - q1,), jnp.int32)
        for w in range(NW):
            k = w // NC
            if w % NC == 0:
                parts.append(lax.dynamic_slice(flat, (k * q0,), (q0,)))
            else:
                parts.append(jnp.concatenate(
                    [lax.dynamic_slice(flat, (f0 + k * q1,), (q1,)), pad1]))
        return jnp.stack(parts).reshape(NW, ncmax, CHUNK, batch_sp)

    src_c = _split(src_f)
    dst_c = _split(dst_f)

    x_pad = jnp.pad(x, ((0, n_pad - n), (0, 0)))
    zero1 = jnp.zeros((n_pad,), jnp.float32)
    zero2 = jnp.zeros((n_pad, f), jnp.float32)

    deg_p = _degree_kernel(n_pad, per_tile_batches)(dst_r, zero1)
    deg3 = deg_p.reshape(NC, n_pad, 1)

    nb = 4
    blk = n_pad // nb
    xw, u1, dis = pl.pallas_call(
        _k2_body,
        grid=(nb,),
        in_specs=[
            pl.BlockSpec((blk, f), lambda i: (i, 0)),
            pl.BlockSpec((f, f), lambda i: (0, 0)),
            pl.BlockSpec((NC, blk, 1), lambda i: (0, i, 0)),
        ],
        out_specs=[
            pl.BlockSpec((blk, f), lambda i: (i, 0)),
            pl.BlockSpec((blk, f), lambda i: (i, 0)),
            pl.BlockSpec((blk, 1), lambda i: (i, 0)),
        ],
        out_shape=[
            jax.ShapeDtypeStruct((n_pad, f), jnp.float32),
            jax.ShapeDtypeStruct((n_pad, f), jnp.float32),
            jax.ShapeDtypeStruct((n_pad, 1), jnp.float32),
        ],
    )(x_pad, W1, deg3)

    tmp1 = _spmm_kernel(n_pad, f, nc0, nc1, batch_sp)(src_c, dst_c, u1, zero2)

    h1, u2 = pl.pallas_call(
        _k4_body,
        grid=(nb,),
        in_specs=[
            pl.BlockSpec((NC, blk, f), lambda i: (0, i, 0)),
            pl.BlockSpec((blk, f), lambda i: (i, 0)),
            pl.BlockSpec((blk, 1), lambda i: (i, 0)),
            pl.BlockSpec((1, f), lambda i: (0, 0)),
        ],
        out_specs=[
            pl.BlockSpec((blk, f), lambda i: (i, 0)),
            pl.BlockSpec((blk, f), lambda i: (i, 0)),
        ],
        out_shape=[
            jax.ShapeDtypeStruct((n_pad, f), jnp.float32),
            jax.ShapeDtypeStruct((n_pad, f), jnp.float32),
        ],
    )(tmp1, xw, dis, b1.reshape(1, f))

    tmp2 = _spmm_kernel(n_pad, f, nc0, nc1, batch_sp)(src_c, dst_c, u2, zero2)

    ob = 2000
    og = n // ob
    out = pl.pallas_call(
        _k6_body,
        grid=(og,),
        in_specs=[
            pl.BlockSpec((NC, ob, f), lambda i: (0, i, 0)),
            pl.BlockSpec((ob, f), lambda i: (i, 0)),
            pl.BlockSpec((ob, 1), lambda i: (i, 0)),
            pl.BlockSpec((f, o), lambda i: (0, 0)),
            pl.BlockSpec((1, o), lambda i: (0, 0)),
        ],
        out_specs=pl.BlockSpec((ob, o), lambda i: (i, 0)),
        out_shape=jax.ShapeDtypeStruct((n, o), jnp.float32),
    )(tmp2, h1, dis, W2, b2.reshape(1, o))

    return out


# speedup vs baseline: 1.2891x; 1.0332x over previous
import functools

import jax
import jax.numpy as jnp
from jax import lax
from jax.experimental import pallas as pl
from jax.experimental.pallas import tpu as pltpu
from jax.experimental.pallas import tpu_sc as plsc

LANES = 128
NC, NS = 2, 16
NW = NC * NS


def _mesh():
    return plsc.VectorSubcoreMesh(core_axis_name="c", subcore_axis_name="s")


def _degree_kernel(n_pad, n_batches):
    @functools.partial(
        pl.kernel,
        mesh=_mesh(),
        out_type=jax.ShapeDtypeStruct((NC, n_pad), jnp.float32),
        scratch_types=[
            pltpu.VMEM((n_batches, LANES), jnp.int32),
            pltpu.VMEM((LANES,), jnp.float32),
            pltpu.VMEM_SHARED((n_pad,), jnp.float32),
        ],
    )
    def k(dst_hbm, zero1_hbm, out_hbm, dst_v, ones_v, acc):
        cid = lax.axis_index("c")
        sid = lax.axis_index("s")
        wid = sid * NC + cid

        @pl.when(sid == 0)
        def _():
            pltpu.sync_copy(zero1_hbm, acc)

        for i in range(LANES // 16):
            ones_v[pl.ds(16 * i, 16)] = jnp.ones((16,), jnp.float32)
        pltpu.sync_copy(dst_hbm.at[wid], dst_v)
        plsc.subcore_barrier()

        def body(b, carry):
            pltpu.sync_copy(ones_v, acc.at[dst_v.at[b]], add=True)
            return carry

        lax.fori_loop(0, n_batches, body, 0)
        plsc.subcore_barrier()

        @pl.when(sid == 0)
        def _():
            pltpu.sync_copy(acc, out_hbm.at[cid])

    return k


CHUNK = 16
NBUF = 4


def _spmm_kernel(n_pad, f, nc0, nc1, batch):
    rows_per = n_pad // NS

    @functools.partial(
        pl.kernel,
        mesh=_mesh(),
        out_type=jax.ShapeDtypeStruct((NC, n_pad, f), jnp.float32),
        scratch_types=[
            pltpu.VMEM((CHUNK, batch), jnp.int32),
            pltpu.VMEM((CHUNK, batch), jnp.int32),
        ] + [pltpu.VMEM((batch, f), jnp.float32) for _ in range(NBUF)]
          + [
            pltpu.VMEM_SHARED((n_pad, f), jnp.float32),
        ] + [pltpu.SemaphoreType.DMA for _ in range(NBUF)],
    )
    def k(src_hbm, dst_hbm, u_hbm, zero_hbm, out_hbm, src_v, dst_v, *rest):
        bufs = rest[:NBUF]
        acc = rest[NBUF]
        sems = rest[NBUF + 1:]
        cid = lax.axis_index("c")
        sid = lax.axis_index("s")
        wid = sid * NC + cid
        sl = pl.ds(sid * rows_per, rows_per)
        n_local = jnp.where(cid == 0, nc0, nc1)

        pltpu.sync_copy(zero_hbm.at[sl], acc.at[sl])
        plsc.subcore_barrier()

        def chunk_body(ci, carry):
            pltpu.sync_copy(src_hbm.at[wid, ci], src_v)
            pltpu.sync_copy(dst_hbm.at[wid, ci], dst_v)
            for j in range(NBUF):
                pltpu.async_copy(u_hbm.at[src_v.at[j]], bufs[j], sems[j])
            for g in range(0, CHUNK, NBUF):
                for j in range(NBUF):
                    b = g + j
                    pltpu.make_async_copy(
                        u_hbm.at[src_v.at[b]], bufs[j], sems[j]).wait()
                    pltpu.sync_copy(bufs[j], acc.at[dst_v.at[b]], add=True)
                    if b + NBUF < CHUNK:
                        pltpu.async_copy(
                            u_hbm.at[src_v.at[b + NBUF]], bufs[j], sems[j])
            return carry

        lax.fori_loop(0, n_local, chunk_body, 0)
        plsc.subcore_barrier()
        pltpu.sync_copy(acc.at[sl], out_hbm.at[cid, sl])

    return k


def _k2_body(x_ref, w_ref, deg_ref, xw_ref, u1_ref, dis_ref):
    xw = jnp.dot(x_ref[...], w_ref[...], preferred_element_type=jnp.float32)
    d = deg_ref[0] + deg_ref[1] + 1.0
    dis = lax.rsqrt(d)
    xw_ref[...] = xw
    u1_ref[...] = xw * dis
    dis_ref[...] = dis


def _k4_body(t_ref, xw_ref, dis_ref, b_ref, h1_ref, u2_ref):
    t = t_ref[0] + t_ref[1]
    dis = dis_ref[...]
    agg = dis * t + (dis * dis) * xw_ref[...]
    h1 = jnp.maximum(agg + b_ref[...], 0.0)
    h1_ref[...] = h1
    u2_ref[...] = dis * h1


def _k6_body(t_ref, h_ref, dis_ref, w_ref, b_ref, o_ref):
    t = t_ref[0] + t_ref[1]
    dis = dis_ref[...]
    agg = dis * t + (dis * dis) * h_ref[...]
    o_ref[...] = (
        jnp.dot(agg, w_ref[...], preferred_element_type=jnp.float32) + b_ref[...]
    )


def kernel(x, edge_index, W1, b1, W2, b2):
    n, f = x.shape
    e = edge_index.shape[1]
    o = W2.shape[1]

    n_pad = 10240
    batch_sp = 64
    e_chunk = CHUNK * batch_sp
    n_chunks = -(-e // (NW * e_chunk))
    e_pad = NW * e_chunk * n_chunks
    per_tile_batches = e_pad // (NW * LANES)

    src = edge_index[0].astype(jnp.int32)
    dst = edge_index[1].astype(jnp.int32)
    fill = n + jnp.arange(e_pad - e, dtype=jnp.int32) % (n_pad - n)
    src_f = jnp.concatenate([src, fill])
    dst_f = jnp.concatenate([dst, fill])
    src_r = src_f.reshape(NW, per_tile_batches, LANES)
    dst_r = dst_f.reshape(NW, per_tile_batches, LANES)

    nc0, nc1 = 10, 10
    assert NS * (nc0 + nc1) * e_chunk >= e
    ncmax = max(nc0, nc1)
    q0, q1 = nc0 * e_chunk, nc1 * e_chunk
    f0 = NS * q0

    def _split(flat):
        parts = []
        pad1 = jnp.zeros((ncmax * e_chunk - q1,), jnp.int32)
        for w in range(NW):
            k = w // NC
            if w % NC == 0:
                parts.append(lax.dynamic_slice(flat, (k * q0,), (q0,)))
            else:
                parts.append(jnp.concatenate(
                    [lax.dynamic_slice(flat, (f0 + k * q1,), (q1,)), pad1]))
        return jnp.stack(parts).reshape(NW, ncmax, CHUNK, batch_sp)

    src_c = _split(src_f)
    dst_c = _split(dst_f)

    x_pad = jnp.pad(x, ((0, n_pad - n), (0, 0)))
    zero1 = jnp.zeros((n_pad,), jnp.float32)
    zero2 = jnp.zeros((n_pad, f), jnp.float32)

    deg_p = _degree_kernel(n_pad, per_tile_batches)(dst_r, zero1)
    deg3 = deg_p.reshape(NC, n_pad, 1)

    nb = 4
    blk = n_pad // nb
    xw, u1, dis = pl.pallas_call(
        _k2_body,
        grid=(nb,),
        in_specs=[
            pl.BlockSpec((blk, f), lambda i: (i, 0)),
            pl.BlockSpec((f, f), lambda i: (0, 0)),
            pl.BlockSpec((NC, blk, 1), lambda i: (0, i, 0)),
        ],
        out_specs=[
            pl.BlockSpec((blk, f), lambda i: (i, 0)),
            pl.BlockSpec((blk, f), lambda i: (i, 0)),
            pl.BlockSpec((blk, 1), lambda i: (i, 0)),
        ],
        out_shape=[
            jax.ShapeDtypeStruct((n_pad, f), jnp.float32),
            jax.ShapeDtypeStruct((n_pad, f), jnp.float32),
            jax.ShapeDtypeStruct((n_pad, 1), jnp.float32),
        ],
    )(x_pad, W1, deg3)

    tmp1 = _spmm_kernel(n_pad, f, nc0, nc1, batch_sp)(src_c, dst_c, u1, zero2)

    h1, u2 = pl.pallas_call(
        _k4_body,
        grid=(nb,),
        in_specs=[
            pl.BlockSpec((NC, blk, f), lambda i: (0, i, 0)),
            pl.BlockSpec((blk, f), lambda i: (i, 0)),
            pl.BlockSpec((blk, 1), lambda i: (i, 0)),
            pl.BlockSpec((1, f), lambda i: (0, 0)),
        ],
        out_specs=[
            pl.BlockSpec((blk, f), lambda i: (i, 0)),
            pl.BlockSpec((blk, f), lambda i: (i, 0)),
        ],
        out_shape=[
            jax.ShapeDtypeStruct((n_pad, f), jnp.float32),
            jax.ShapeDtypeStruct((n_pad, f), jnp.float32),
        ],
    )(tmp1, xw, dis, b1.reshape(1, f))

    tmp2 = _spmm_kernel(n_pad, f, nc0, nc1, batch_sp)(src_c, dst_c, u2, zero2)

    ob = 2000
    og = n // ob
    out = pl.pallas_call(
        _k6_body,
        grid=(og,),
        in_specs=[
            pl.BlockSpec((NC, ob, f), lambda i: (0, i, 0)),
            pl.BlockSpec((ob, f), lambda i: (i, 0)),
            pl.BlockSpec((ob, 1), lambda i: (i, 0)),
            pl.BlockSpec((f, o), lambda i: (0, 0)),
            pl.BlockSpec((1, o), lambda i: (0, 0)),
        ],
        out_specs=pl.BlockSpec((ob, o), lambda i: (i, 0)),
        out_shape=jax.ShapeDtypeStruct((n, o), jnp.float32),
    )(tmp2, h1, dis, W2, b2.reshape(1, o))

    return out
